# Initial kernel scaffold; baseline (speedup 1.0000x reference)
import jax
import jax.numpy as jnp
from jax.experimental import pallas as pl


def kernel(query_x, phase_state, chase_state, wicket_buffer, player_table, W_enc, b_enc, W_msg, W_film, b_film, W_m, b_m, W_g1, b_g1, W_g2, b_g2, W_q, b_q, W_c, b_c, W_b1, b_b1, W_b2, b_b2, W_w1, b_w1, W_w2, b_w2, striker_ids, bowler_ids, nonstriker_ids, edge_index):
    raise NotImplementedError("write your pallas kernel here")


# speedup vs baseline: 3.9799x; 3.9799x over previous
import functools

import jax
import jax.numpy as jnp
from jax import lax
from jax.experimental import pallas as pl
from jax.experimental.pallas import tpu as pltpu
from jax.experimental.pallas import tpu_sc as plsc

N_NODES = 10000
N_PAD = 10240
N_EDGES = 320000
B = 4096
H = 128
PED = 64

NC = 2
NS = 16
NW = NC * NS
EDGES_PER_W = N_EDGES // NW
CHUNK = 80
NCHUNK = EDGES_PER_W // CHUNK
ROWS_PER_S = N_PAD // NS
DEGW = 16

_f32 = jnp.float32


def _ln(x, eps=1e-5):
    mu = jnp.mean(x, axis=-1, keepdims=True)
    var = jnp.var(x, axis=-1, keepdims=True)
    return (x - mu) / jnp.sqrt(var + eps)


_RB = 2048


def _encode_body(pt_ref, we_ref, be_ref, w0_ref, h_ref, hw0_ref):
    x = pt_ref[...] @ we_ref[...] + be_ref[...]
    h = jax.nn.gelu(_ln(x))
    h_ref[...] = h
    hw0_ref[...] = h @ w0_ref[...]


def _encode(pt, we, be, w0):
    return pl.pallas_call(
        _encode_body,
        grid=(N_PAD // _RB,),
        in_specs=[
            pl.BlockSpec((_RB, PED), lambda i: (i, 0)),
            pl.BlockSpec((PED, H), lambda i: (0, 0)),
            pl.BlockSpec((1, H), lambda i: (0, 0)),
            pl.BlockSpec((H, H), lambda i: (0, 0)),
        ],
        out_specs=[
            pl.BlockSpec((_RB, H), lambda i: (i, 0)),
            pl.BlockSpec((_RB, H), lambda i: (i, 0)),
        ],
        out_shape=[
            jax.ShapeDtypeStruct((N_PAD, H), _f32),
            jax.ShapeDtypeStruct((N_PAD, H), _f32),
        ],
    )(pt, we, be, w0)


def _film_body(ph_ref, ch_ref, wk_ref, wf_ref, bf_ref, out_ref):
    cond = jnp.concatenate([ph_ref[...], ch_ref[...], wk_ref[...]], axis=1)
    f = jnp.sum(cond, axis=0, keepdims=True) * (1.0 / B)
    out_ref[...] = f @ wf_ref[...] + bf_ref[...]


def _film(ph, ch, wk, wf, bf):
    return pl.pallas_call(
        _film_body,
        out_shape=jax.ShapeDtypeStruct((1, 2 * H), _f32),
    )(ph, ch, wk, wf, bf)


def _update_body(with_next, h_ref, p0_ref, p1_ref, d0_ref, d1_ref, gb_ref,
                 wn_ref, hn_ref, hwn_ref=None):
    deg = jnp.maximum(d0_ref[...][:, :1] + d1_ref[...][:, :1], 1.0)
    agg = (p0_ref[...] + p1_ref[...]) / deg
    gb = gb_ref[...]
    gamma = gb[:, :H]
    beta = gb[:, H:]
    u = jax.nn.gelu(agg * (1.0 + gamma) + beta)
    hn = _ln(h_ref[...] + u)
    hn_ref[...] = hn
    if with_next:
        hwn_ref[...] = hn @ wn_ref[...]


def _update(h, p0, p1, d0, d1, gb, wn):
    with_next = wn is not None
    if not with_next:
        wn = jnp.zeros((H, H), _f32)
    body = functools.partial(_update_body, with_next)
    out_specs = [pl.BlockSpec((_RB, H), lambda i: (i, 0))]
    out_shape = [jax.ShapeDtypeStruct((N_PAD, H), _f32)]
    if with_next:
        out_specs.append(pl.BlockSpec((_RB, H), lambda i: (i, 0)))
        out_shape.append(jax.ShapeDtypeStruct((N_PAD, H), _f32))
    return pl.pallas_call(
        body,
        grid=(N_PAD // _RB,),
        in_specs=[
            pl.BlockSpec((_RB, H), lambda i: (i, 0)),
            pl.BlockSpec((_RB, H), lambda i: (i, 0)),
            pl.BlockSpec((_RB, H), lambda i: (i, 0)),
            pl.BlockSpec((_RB, H), lambda i: (i, 0)),
            pl.BlockSpec((_RB, H), lambda i: (i, 0)),
            pl.BlockSpec((1, 2 * H), lambda i: (0, 0)),
            pl.BlockSpec((H, H), lambda i: (0, 0)),
        ],
        out_specs=out_specs,
        out_shape=out_shape,
    )(h, p0, p1, d0, d1, gb, wn)


_QB = 512


def _head_body(st_ref, bo_ref, ns_ref, qx_ref, wm_ref, bm_ref, wg1_ref,
               bg1_ref, wg2_ref, bg2_ref, wq_ref, bq_ref, wc_ref, bc_ref,
               wbw1_ref, bbw1_ref, wbw2_ref, bbw2_ref, out_ref):
    st = st_ref[...]
    bo = bo_ref[...]
    ns = ns_ref[...]
    wm = wm_ref[...]
    base = jax.nn.gelu(_ln(st @ wm[:H] + bo @ wm[H:] + bm_ref[...]))
    gate = jax.nn.sigmoid(
        jax.nn.gelu(ns @ wg1_ref[...] + bg1_ref[...]) @ wg2_ref[...]
        + bg2_ref[...])
    matchup = base * (1.0 + 0.1 * gate)
    query = jax.nn.gelu(_ln(qx_ref[...] @ wq_ref[...] + bq_ref[...]))
    wc = wc_ref[...]
    readout = jax.nn.gelu(_ln(matchup @ wc[:H] + query @ wc[H:] + bc_ref[...]))
    t = jax.nn.relu(readout @ wbw1_ref[...] + bbw1_ref[...])
    out_ref[...] = t @ wbw2_ref[...] + bbw2_ref[...]


def _head(st, bo, ns, qx, wm, bm, wg1, bg1, wg2, bg2, wq, bq, wc, bc,
          wbw1, bbw1, wbw2, bbw2):
    row = lambda i: (i, 0)
    full = lambda i: (0, 0)
    return pl.pallas_call(
        _head_body,
        grid=(B // _QB,),
        in_specs=[
            pl.BlockSpec((_QB, H), row),
            pl.BlockSpec((_QB, H), row),
            pl.BlockSpec((_QB, H), row),
            pl.BlockSpec((_QB, H), row),
            pl.BlockSpec((2 * H, H), full),
            pl.BlockSpec((1, H), full),
            pl.BlockSpec((H, H // 4), full),
            pl.BlockSpec((1, H // 4), full),
            pl.BlockSpec((H // 4, H), full),
            pl.BlockSpec((1, H), full),
            pl.BlockSpec((H, H), full),
            pl.BlockSpec((1, H), full),
            pl.BlockSpec((2 * H, H), full),
            pl.BlockSpec((1, H), full),
            pl.BlockSpec((H, H), full),
            pl.BlockSpec((1, H), full),
            pl.BlockSpec((H, 2), full),
            pl.BlockSpec((1, 2), full),
        ],
        out_specs=pl.BlockSpec((_QB, 2), row),
        out_shape=jax.ShapeDtypeStruct((B, 2), _f32),
    )(st, bo, ns, qx, wm, bm, wg1, bg1, wg2, bg2, wq, bq, wc, bc,
      wbw1, bbw1, wbw2, bbw2)


_mesh = plsc.VectorSubcoreMesh(core_axis_name="c", subcore_axis_name="s",
                               num_cores=NC, num_subcores=NS)


_EDGE_SCRATCH = [
    pltpu.VMEM_SHARED((N_PAD, H), _f32),
    pltpu.VMEM((CHUNK,), jnp.int32),
    pltpu.VMEM((CHUNK,), jnp.int32),
    pltpu.VMEM((CHUNK, H), _f32),
    pltpu.SemaphoreType.DMA,
]


_NZB = ROWS_PER_S // CHUNK


def _deg_body(dst_hbm, ones_hbm, zrow_hbm, out_hbm,
              agg_sh, src_v, dst_v, rows_v, sem):
    c = lax.axis_index("c")
    s = lax.axis_index("s")
    w = s * NC + c
    r0 = s * ROWS_PER_S
    pltpu.sync_copy(zrow_hbm, rows_v)
    for k in range(_NZB):
        pltpu.sync_copy(rows_v, agg_sh.at[pl.ds(r0 + k * CHUNK, CHUNK)])
    pltpu.sync_copy(ones_hbm, rows_v)
    plsc.subcore_barrier()

    def step(i, carry):
        base = pl.multiple_of(w * EDGES_PER_W + i * CHUNK, 8)
        pltpu.sync_copy(dst_hbm.at[pl.ds(base, CHUNK)], dst_v)
        pltpu.sync_copy(rows_v, agg_sh.at[dst_v], add=True)
        return carry

    lax.fori_loop(0, NCHUNK, step, 0)
    plsc.subcore_barrier()
    for k in range(_NZB):
        pltpu.sync_copy(agg_sh.at[pl.ds(r0 + k * CHUNK, CHUNK)], rows_v)
        pltpu.sync_copy(rows_v, out_hbm.at[c, pl.ds(r0 + k * CHUNK, CHUNK)])


def _edge_body(hw_hbm, src_hbm, dst_hbm, zrow_hbm, out_hbm,
               agg_sh, src_v, dst_v, rows_v, sem):
    c = lax.axis_index("c")
    s = lax.axis_index("s")
    w = s * NC + c
    r0 = s * ROWS_PER_S
    pltpu.sync_copy(zrow_hbm, rows_v)
    for k in range(_NZB):
        pltpu.sync_copy(rows_v, agg_sh.at[pl.ds(r0 + k * CHUNK, CHUNK)])
    plsc.subcore_barrier()

    def step(i, carry):
        base = pl.multiple_of(w * EDGES_PER_W + i * CHUNK, 8)
        pltpu.sync_copy(src_hbm.at[pl.ds(base, CHUNK)], src_v)
        pltpu.sync_copy(dst_hbm.at[pl.ds(base, CHUNK)], dst_v)
        pltpu.async_copy(hw_hbm.at[src_v], rows_v, sem).wait()
        pltpu.sync_copy(rows_v, agg_sh.at[dst_v], add=True)
        return carry

    lax.fori_loop(0, NCHUNK, step, 0)
    plsc.subcore_barrier()
    for k in range(_NZB):
        pltpu.sync_copy(agg_sh.at[pl.ds(r0 + k * CHUNK, CHUNK)], rows_v)
        pltpu.sync_copy(rows_v, out_hbm.at[c, pl.ds(r0 + k * CHUNK, CHUNK)])


_deg = pl.kernel(
    _deg_body,
    out_type=jax.ShapeDtypeStruct((NC, N_PAD, H), _f32),
    mesh=_mesh,
    scratch_types=_EDGE_SCRATCH,
)

_edge = pl.kernel(
    _edge_body,
    out_type=jax.ShapeDtypeStruct((NC, N_PAD, H), _f32),
    mesh=_mesh,
    scratch_types=_EDGE_SCRATCH,
)


_GPW = (3 * B) // NW
_GCH = 128


def _gather_body(h_hbm, ids_hbm, out_hbm, idx_v, rows_v, sem):
    c = lax.axis_index("c")
    s = lax.axis_index("s")
    w = s * NC + c
    for j in range(_GPW // _GCH):
        base = pl.multiple_of(w * _GPW + j * _GCH, 8)
        pltpu.sync_copy(ids_hbm.at[pl.ds(base, _GCH)], idx_v)
        pltpu.async_copy(h_hbm.at[idx_v], rows_v, sem).wait()
        pltpu.sync_copy(rows_v, out_hbm.at[pl.ds(base, _GCH)])


_gather = pl.kernel(
    _gather_body,
    out_type=jax.ShapeDtypeStruct((3 * B, H), _f32),
    mesh=_mesh,
    scratch_types=[
        pltpu.VMEM((_GCH,), jnp.int32),
        pltpu.VMEM((_GCH, H), _f32),
        pltpu.SemaphoreType.DMA,
    ],
)


def kernel(query_x, phase_state, chase_state, wicket_buffer, player_table,
           W_enc, b_enc, W_msg, W_film, b_film, W_m, b_m, W_g1, b_g1, W_g2,
           b_g2, W_q, b_q, W_c, b_c, W_b1, b_b1, W_b2, b_b2, W_w1, b_w1,
           W_w2, b_w2, striker_ids, bowler_ids, nonstriker_ids, edge_index):
    src = edge_index[0]
    dst = edge_index[1]

    gb = _film(phase_state, chase_state, wicket_buffer, W_film,
               b_film.reshape(1, -1))
    pt_pad = jnp.pad(player_table, ((0, N_PAD - N_NODES), (0, 0)))
    h, hw = _encode(pt_pad, W_enc, b_enc.reshape(1, -1), W_msg[0])

    zrow = jnp.zeros((CHUNK, H), _f32)
    ones = jnp.ones((CHUNK, H), _f32)

    degparts = _deg(dst, ones, zrow)
    d0, d1 = degparts[0], degparts[1]
    parts = _edge(hw, src, dst, zrow)
    h, hw = _update(h, parts[0], parts[1], d0, d1, gb, W_msg[1])
    parts = _edge(hw, src, dst, zrow)
    h, hw = _update(h, parts[0], parts[1], d0, d1, gb, W_msg[2])
    parts = _edge(hw, src, dst, zrow)
    h = _update(h, parts[0], parts[1], d0, d1, gb, None)[0]

    ids = jnp.concatenate([striker_ids, bowler_ids, nonstriker_ids])
    gath = _gather(h, ids.astype(jnp.int32))
    st, bo, ns = gath[:B], gath[B:2 * B], gath[2 * B:]

    wbw1 = jnp.concatenate([W_b1, W_w1], axis=1)
    bbw1 = jnp.concatenate([b_b1, b_w1]).reshape(1, -1)
    wbw2 = jnp.zeros((H, 2), _f32)
    wbw2 = wbw2.at[:H // 2, 0:1].set(W_b2)
    wbw2 = wbw2.at[H // 2:, 1:2].set(W_w2)
    bbw2 = jnp.concatenate([b_b2, b_w2]).reshape(1, -1)

    return _head(st, bo, ns, query_x, W_m, b_m.reshape(1, -1),
                 W_g1, b_g1.reshape(1, -1), W_g2, b_g2.reshape(1, -1),
                 W_q, b_q.reshape(1, -1), W_c, b_c.reshape(1, -1),
                 wbw1, bbw1, wbw2, bbw2)


# speedup vs baseline: 7.8596x; 1.9748x over previous
import functools

import jax
import jax.numpy as jnp
from jax import lax
from jax.experimental import pallas as pl
from jax.experimental.pallas import tpu as pltpu
from jax.experimental.pallas import tpu_sc as plsc

N_NODES = 10000
N_PAD = 10240
N_EDGES = 320000
B = 4096
H = 128
PED = 64

NC = 2
NS = 16
NW = NC * NS
EDGES_PER_W = N_EDGES // NW
CHUNK = 80
NCHUNK = EDGES_PER_W // CHUNK
PF = 25
NBLK = NCHUNK // PF
ROWS_PER_S = N_PAD // NS
DEGW = 16

_f32 = jnp.float32


def _ln(x, eps=1e-5):
    mu = jnp.mean(x, axis=-1, keepdims=True)
    var = jnp.var(x, axis=-1, keepdims=True)
    return (x - mu) / jnp.sqrt(var + eps)


_RB = 2048


def _encode_body(pt_ref, we_ref, be_ref, w0_ref, h_ref, hw0_ref):
    x = pt_ref[...] @ we_ref[...] + be_ref[...]
    h = jax.nn.gelu(_ln(x))
    h_ref[...] = h
    hw0_ref[...] = h @ w0_ref[...]


def _encode(pt, we, be, w0):
    return pl.pallas_call(
        _encode_body,
        grid=(N_PAD // _RB,),
        in_specs=[
            pl.BlockSpec((_RB, PED), lambda i: (i, 0)),
            pl.BlockSpec((PED, H), lambda i: (0, 0)),
            pl.BlockSpec((1, H), lambda i: (0, 0)),
            pl.BlockSpec((H, H), lambda i: (0, 0)),
        ],
        out_specs=[
            pl.BlockSpec((_RB, H), lambda i: (i, 0)),
            pl.BlockSpec((_RB, H), lambda i: (i, 0)),
        ],
        out_shape=[
            jax.ShapeDtypeStruct((N_PAD, H), _f32),
            jax.ShapeDtypeStruct((N_PAD, H), _f32),
        ],
    )(pt, we, be, w0)


def _film_body(ph_ref, ch_ref, wk_ref, wf_ref, bf_ref, out_ref):
    cond = jnp.concatenate([ph_ref[...], ch_ref[...], wk_ref[...]], axis=1)
    f = jnp.sum(cond, axis=0, keepdims=True) * (1.0 / B)
    out_ref[...] = f @ wf_ref[...] + bf_ref[...]


def _film(ph, ch, wk, wf, bf):
    return pl.pallas_call(
        _film_body,
        out_shape=jax.ShapeDtypeStruct((1, 2 * H), _f32),
    )(ph, ch, wk, wf, bf)


def _update_body(with_next, h_ref, p0_ref, p1_ref, d0_ref, d1_ref, gb_ref,
                 wn_ref, hn_ref, hwn_ref=None):
    deg = jnp.maximum(d0_ref[...][:, :1] + d1_ref[...][:, :1], 1.0)
    agg = (p0_ref[...] + p1_ref[...]) / deg
    gb = gb_ref[...]
    gamma = gb[:, :H]
    beta = gb[:, H:]
    u = jax.nn.gelu(agg * (1.0 + gamma) + beta)
    hn = _ln(h_ref[...] + u)
    hn_ref[...] = hn
    if with_next:
        hwn_ref[...] = hn @ wn_ref[...]


def _update(h, p0, p1, d0, d1, gb, wn):
    with_next = wn is not None
    if not with_next:
        wn = jnp.zeros((H, H), _f32)
    body = functools.partial(_update_body, with_next)
    out_specs = [pl.BlockSpec((_RB, H), lambda i: (i, 0))]
    out_shape = [jax.ShapeDtypeStruct((N_PAD, H), _f32)]
    if with_next:
        out_specs.append(pl.BlockSpec((_RB, H), lambda i: (i, 0)))
        out_shape.append(jax.ShapeDtypeStruct((N_PAD, H), _f32))
    return pl.pallas_call(
        body,
        grid=(N_PAD // _RB,),
        in_specs=[
            pl.BlockSpec((_RB, H), lambda i: (i, 0)),
            pl.BlockSpec((_RB, H), lambda i: (i, 0)),
            pl.BlockSpec((_RB, H), lambda i: (i, 0)),
            pl.BlockSpec((_RB, H), lambda i: (i, 0)),
            pl.BlockSpec((_RB, H), lambda i: (i, 0)),
            pl.BlockSpec((1, 2 * H), lambda i: (0, 0)),
            pl.BlockSpec((H, H), lambda i: (0, 0)),
        ],
        out_specs=out_specs,
        out_shape=out_shape,
    )(h, p0, p1, d0, d1, gb, wn)


_QB = 512


def _head_body(st_ref, bo_ref, ns_ref, qx_ref, wm_ref, bm_ref, wg1_ref,
               bg1_ref, wg2_ref, bg2_ref, wq_ref, bq_ref, wc_ref, bc_ref,
               wbw1_ref, bbw1_ref, wbw2_ref, bbw2_ref, out_ref):
    st = st_ref[...]
    bo = bo_ref[...]
    ns = ns_ref[...]
    wm = wm_ref[...]
    base = jax.nn.gelu(_ln(st @ wm[:H] + bo @ wm[H:] + bm_ref[...]))
    gate = jax.nn.sigmoid(
        jax.nn.gelu(ns @ wg1_ref[...] + bg1_ref[...]) @ wg2_ref[...]
        + bg2_ref[...])
    matchup = base * (1.0 + 0.1 * gate)
    query = jax.nn.gelu(_ln(qx_ref[...] @ wq_ref[...] + bq_ref[...]))
    wc = wc_ref[...]
    readout = jax.nn.gelu(_ln(matchup @ wc[:H] + query @ wc[H:] + bc_ref[...]))
    t = jax.nn.relu(readout @ wbw1_ref[...] + bbw1_ref[...])
    out_ref[...] = t @ wbw2_ref[...] + bbw2_ref[...]


def _head(st, bo, ns, qx, wm, bm, wg1, bg1, wg2, bg2, wq, bq, wc, bc,
          wbw1, bbw1, wbw2, bbw2):
    row = lambda i: (i, 0)
    full = lambda i: (0, 0)
    return pl.pallas_call(
        _head_body,
        grid=(B // _QB,),
        in_specs=[
            pl.BlockSpec((_QB, H), row),
            pl.BlockSpec((_QB, H), row),
            pl.BlockSpec((_QB, H), row),
            pl.BlockSpec((_QB, H), row),
            pl.BlockSpec((2 * H, H), full),
            pl.BlockSpec((1, H), full),
            pl.BlockSpec((H, H // 4), full),
            pl.BlockSpec((1, H // 4), full),
            pl.BlockSpec((H // 4, H), full),
            pl.BlockSpec((1, H), full),
            pl.BlockSpec((H, H), full),
            pl.BlockSpec((1, H), full),
            pl.BlockSpec((2 * H, H), full),
            pl.BlockSpec((1, H), full),
            pl.BlockSpec((H, H), full),
            pl.BlockSpec((1, H), full),
            pl.BlockSpec((H, 2), full),
            pl.BlockSpec((1, 2), full),
        ],
        out_specs=pl.BlockSpec((_QB, 2), row),
        out_shape=jax.ShapeDtypeStruct((B, 2), _f32),
    )(st, bo, ns, qx, wm, bm, wg1, bg1, wg2, bg2, wq, bq, wc, bc,
      wbw1, bbw1, wbw2, bbw2)


_mesh = plsc.VectorSubcoreMesh(core_axis_name="c", subcore_axis_name="s",
                               num_cores=NC, num_subcores=NS)


_NZB = ROWS_PER_S // CHUNK


def _edge_scratch(width):
    return [
        pltpu.VMEM_SHARED((N_PAD, width), _f32),
        pltpu.VMEM((PF, CHUNK), jnp.int32),
        pltpu.VMEM((PF, CHUNK), jnp.int32),
        pltpu.VMEM((CHUNK, width), _f32),
        pltpu.VMEM((CHUNK, width), _f32),
        pltpu.SemaphoreType.DMA,
        pltpu.SemaphoreType.DMA,
    ]


def _make_edge(width):
    def body(hw_hbm, src3_hbm, dst3_hbm, zrow_hbm, out_hbm,
             agg_sh, srcs_v, dsts_v, rows0, rows1, sem0, sem1):
        c = lax.axis_index("c")
        s = lax.axis_index("s")
        w = s * NC + c
        r0 = s * ROWS_PER_S
        pltpu.sync_copy(zrow_hbm, rows0)
        for k in range(_NZB):
            pltpu.sync_copy(rows0, agg_sh.at[pl.ds(r0 + k * CHUNK, CHUNK)])
        plsc.subcore_barrier()
        rows = (rows0, rows1)
        sems = (sem0, sem1)

        def block(p, carry):
            pltpu.sync_copy(src3_hbm.at[w, p], srcs_v)
            pltpu.sync_copy(dst3_hbm.at[w, p], dsts_v)
            pltpu.async_copy(hw_hbm.at[srcs_v.at[0]], rows0, sem0)

            @pl.loop(0, PF - 1, step=2)
            def _pair(j2):
                for b in range(2):
                    j = j2 + b
                    pltpu.async_copy(hw_hbm.at[srcs_v.at[j + 1]],
                                     rows[1 - b], sems[1 - b])
                    pltpu.make_async_copy(hw_hbm.at[srcs_v.at[j]],
                                          rows[b], sems[b]).wait()
                    pltpu.sync_copy(rows[b], agg_sh.at[dsts_v.at[j]],
                                    add=True)

            pltpu.make_async_copy(hw_hbm.at[srcs_v.at[PF - 1]],
                                  rows0, sem0).wait()
            pltpu.sync_copy(rows0, agg_sh.at[dsts_v.at[PF - 1]], add=True)
            return carry

        lax.fori_loop(0, NBLK, block, 0)
        plsc.subcore_barrier()
        for k in range(_NZB):
            pltpu.sync_copy(agg_sh.at[pl.ds(r0 + k * CHUNK, CHUNK)], rows0)
            pltpu.sync_copy(rows0, out_hbm.at[c, pl.ds(r0 + k * CHUNK, CHUNK)])

    return pl.kernel(
        body,
        out_type=jax.ShapeDtypeStruct((NC, N_PAD, width), _f32),
        mesh=_mesh,
        scratch_types=_edge_scratch(width),
    )


_edge = _make_edge(H)


def _deg_body(dst3_hbm, ones_hbm, zrow_hbm, out_hbm,
              agg_sh, dsts_v, rows_v, sem):
    c = lax.axis_index("c")
    s = lax.axis_index("s")
    w = s * NC + c
    r0 = s * ROWS_PER_S
    pltpu.sync_copy(zrow_hbm, rows_v)
    for k in range(_NZB):
        pltpu.sync_copy(rows_v, agg_sh.at[pl.ds(r0 + k * CHUNK, CHUNK)])
    pltpu.sync_copy(ones_hbm, rows_v)
    plsc.subcore_barrier()

    def block(p, carry):
        pltpu.sync_copy(dst3_hbm.at[w, p], dsts_v)

        @pl.loop(0, PF)
        def _step(i):
            pltpu.sync_copy(rows_v, agg_sh.at[dsts_v.at[i]], add=True)

        return carry

    lax.fori_loop(0, NBLK, block, 0)
    plsc.subcore_barrier()
    for k in range(_NZB):
        pltpu.sync_copy(agg_sh.at[pl.ds(r0 + k * CHUNK, CHUNK)], rows_v)
        pltpu.sync_copy(rows_v, out_hbm.at[c, pl.ds(r0 + k * CHUNK, CHUNK)])


_deg = pl.kernel(
    _deg_body,
    out_type=jax.ShapeDtypeStruct((NC, N_PAD, H), _f32),
    mesh=_mesh,
    scratch_types=[
        pltpu.VMEM_SHARED((N_PAD, H), _f32),
        pltpu.VMEM((PF, CHUNK), jnp.int32),
        pltpu.VMEM((CHUNK, H), _f32),
        pltpu.SemaphoreType.DMA,
    ],
)


_GPW = (3 * B) // NW
_GCH = 128


def _gather_body(h_hbm, ids_hbm, out_hbm, idx_v, rows_v, sem):
    c = lax.axis_index("c")
    s = lax.axis_index("s")
    w = s * NC + c
    for j in range(_GPW // _GCH):
        base = pl.multiple_of(w * _GPW + j * _GCH, 8)
        pltpu.sync_copy(ids_hbm.at[pl.ds(base, _GCH)], idx_v)
        pltpu.async_copy(h_hbm.at[idx_v], rows_v, sem).wait()
        pltpu.sync_copy(rows_v, out_hbm.at[pl.ds(base, _GCH)])


_gather = pl.kernel(
    _gather_body,
    out_type=jax.ShapeDtypeStruct((3 * B, H), _f32),
    mesh=_mesh,
    scratch_types=[
        pltpu.VMEM((_GCH,), jnp.int32),
        pltpu.VMEM((_GCH, H), _f32),
        pltpu.SemaphoreType.DMA,
    ],
)


def kernel(query_x, phase_state, chase_state, wicket_buffer, player_table,
           W_enc, b_enc, W_msg, W_film, b_film, W_m, b_m, W_g1, b_g1, W_g2,
           b_g2, W_q, b_q, W_c, b_c, W_b1, b_b1, W_b2, b_b2, W_w1, b_w1,
           W_w2, b_w2, striker_ids, bowler_ids, nonstriker_ids, edge_index):
    src = edge_index[0]
    dst = edge_index[1]

    gb = _film(phase_state, chase_state, wicket_buffer, W_film,
               b_film.reshape(1, -1))
    pt_pad = jnp.pad(player_table, ((0, N_PAD - N_NODES), (0, 0)))
    h, hw = _encode(pt_pad, W_enc, b_enc.reshape(1, -1), W_msg[0])

    zrow = jnp.zeros((CHUNK, H), _f32)
    ones = jnp.ones((CHUNK, H), _f32)

    src3 = src.reshape(NW, NBLK, PF, CHUNK)
    dst3 = dst.reshape(NW, NBLK, PF, CHUNK)
    degparts = _deg(dst3, ones, zrow)
    d0, d1 = degparts[0], degparts[1]
    parts = _edge(hw, src3, dst3, zrow)
    h, hw = _update(h, parts[0], parts[1], d0, d1, gb, W_msg[1])
    parts = _edge(hw, src3, dst3, zrow)
    h, hw = _update(h, parts[0], parts[1], d0, d1, gb, W_msg[2])
    parts = _edge(hw, src3, dst3, zrow)
    h = _update(h, parts[0], parts[1], d0, d1, gb, None)[0]

    ids = jnp.concatenate([striker_ids, bowler_ids, nonstriker_ids])
    gath = _gather(h, ids.astype(jnp.int32))
    st, bo, ns = gath[:B], gath[B:2 * B], gath[2 * B:]

    wbw1 = jnp.concatenate([W_b1, W_w1], axis=1)
    bbw1 = jnp.concatenate([b_b1, b_w1]).reshape(1, -1)
    wbw2 = jnp.zeros((H, 2), _f32)
    wbw2 = wbw2.at[:H // 2, 0:1].set(W_b2)
    wbw2 = wbw2.at[H // 2:, 1:2].set(W_w2)
    bbw2 = jnp.concatenate([b_b2, b_w2]).reshape(1, -1)

    return _head(st, bo, ns, query_x, W_m, b_m.reshape(1, -1),
                 W_g1, b_g1.reshape(1, -1), W_g2, b_g2.reshape(1, -1),
                 W_q, b_q.reshape(1, -1), W_c, b_c.reshape(1, -1),
                 wbw1, bbw1, wbw2, bbw2)


# speedup vs baseline: 7.8643x; 1.0006x over previous
import functools

import jax
import jax.numpy as jnp
from jax import lax
from jax.experimental import pallas as pl
from jax.experimental.pallas import tpu as pltpu
from jax.experimental.pallas import tpu_sc as plsc

N_NODES = 10000
N_PAD = 10240
N_EDGES = 320000
B = 4096
H = 128
PED = 64

NC = 2
NS = 16
NW = NC * NS
EDGES_PER_W = N_EDGES // NW
CHUNK = 80
NCHUNK = EDGES_PER_W // CHUNK
PF = 25
NBLK = NCHUNK // PF
ROWS_PER_S = N_PAD // NS
HD = 144

_f32 = jnp.float32


def _ln(x, eps=1e-5):
    mu = jnp.mean(x, axis=-1, keepdims=True)
    var = jnp.var(x, axis=-1, keepdims=True)
    return (x - mu) / jnp.sqrt(var + eps)


_RB = 2048


def _encode_body(pt_ref, we_ref, be_ref, w0_ref, ph_ref, ch_ref, wk_ref,
                 wf_ref, bf_ref, h_ref, hw0_ref, gb_ref):
    x = pt_ref[...] @ we_ref[...] + be_ref[...]
    h = jax.nn.gelu(_ln(x))
    h_ref[...] = h
    hw0_ref[...] = h @ w0_ref[...]
    cond = jnp.concatenate([ph_ref[...], ch_ref[...], wk_ref[...]], axis=1)
    f = jnp.sum(cond, axis=0, keepdims=True) * (1.0 / B)
    gb_ref[...] = f @ wf_ref[...] + bf_ref[...]


def _encode(pt, we, be, w0, ph, ch, wk, wf, bf):
    return pl.pallas_call(
        _encode_body,
        grid=(N_PAD // _RB,),
        in_specs=[
            pl.BlockSpec((_RB, PED), lambda i: (i, 0)),
            pl.BlockSpec((PED, H), lambda i: (0, 0)),
            pl.BlockSpec((1, H), lambda i: (0, 0)),
            pl.BlockSpec((H, H), lambda i: (0, 0)),
            pl.BlockSpec((B, 6), lambda i: (0, 0)),
            pl.BlockSpec((B, 7), lambda i: (0, 0)),
            pl.BlockSpec((B, 2), lambda i: (0, 0)),
            pl.BlockSpec((15, 2 * H), lambda i: (0, 0)),
            pl.BlockSpec((1, 2 * H), lambda i: (0, 0)),
        ],
        out_specs=[
            pl.BlockSpec((_RB, H), lambda i: (i, 0)),
            pl.BlockSpec((_RB, H), lambda i: (i, 0)),
            pl.BlockSpec((1, 2 * H), lambda i: (0, 0)),
        ],
        out_shape=[
            jax.ShapeDtypeStruct((N_PAD, H), _f32),
            jax.ShapeDtypeStruct((N_PAD, H), _f32),
            jax.ShapeDtypeStruct((1, 2 * H), _f32),
        ],
    )(pt, we, be, w0, ph, ch, wk, wf, bf)


def _update1_body(h_ref, p_ref, dp_ref, gb_ref, wn_ref,
                  hn_ref, hwn_ref, dg_ref):
    deg = jnp.maximum(dp_ref[0, :, :1] + dp_ref[1, :, :1], 1.0)
    agg = (p_ref[0] + p_ref[1]) / deg
    gb = gb_ref[...]
    u = jax.nn.gelu(agg * (1.0 + gb[:, :H]) + gb[:, H:])
    hn = _ln(h_ref[...] + u)
    hn_ref[...] = hn
    hwn_ref[...] = hn @ wn_ref[...]
    dg_ref[...] = jnp.broadcast_to(deg, (_RB, 8))


def _update1(h, parts, degparts, gb, wn):
    return pl.pallas_call(
        _update1_body,
        grid=(N_PAD // _RB,),
        in_specs=[
            pl.BlockSpec((_RB, H), lambda i: (i, 0)),
            pl.BlockSpec((NC, _RB, H), lambda i: (0, i, 0)),
            pl.BlockSpec((NC, _RB, DW), lambda i: (0, i, 0)),
            pl.BlockSpec((1, 2 * H), lambda i: (0, 0)),
            pl.BlockSpec((H, H), lambda i: (0, 0)),
        ],
        out_specs=[
            pl.BlockSpec((_RB, H), lambda i: (i, 0)),
            pl.BlockSpec((_RB, H), lambda i: (i, 0)),
            pl.BlockSpec((_RB, 8), lambda i: (i, 0)),
        ],
        out_shape=[
            jax.ShapeDtypeStruct((N_PAD, H), _f32),
            jax.ShapeDtypeStruct((N_PAD, H), _f32),
            jax.ShapeDtypeStruct((N_PAD, 8), _f32),
        ],
    )(h, parts, degparts, gb, wn)


def _update_body(with_next, h_ref, p_ref, dg_ref, gb_ref,
                 wn_ref, hn_ref, hwn_ref=None):
    agg = (p_ref[0] + p_ref[1]) / dg_ref[...][:, :1]
    gb = gb_ref[...]
    u = jax.nn.gelu(agg * (1.0 + gb[:, :H]) + gb[:, H:])
    hn = _ln(h_ref[...] + u)
    hn_ref[...] = hn
    if with_next:
        hwn_ref[...] = hn @ wn_ref[...]


def _update(h, parts, dg, gb, wn):
    with_next = wn is not None
    if not with_next:
        wn = jnp.zeros((H, H), _f32)
    body = functools.partial(_update_body, with_next)
    out_specs = [pl.BlockSpec((_RB, H), lambda i: (i, 0))]
    out_shape = [jax.ShapeDtypeStruct((N_PAD, H), _f32)]
    if with_next:
        out_specs.append(pl.BlockSpec((_RB, H), lambda i: (i, 0)))
        out_shape.append(jax.ShapeDtypeStruct((N_PAD, H), _f32))
    return pl.pallas_call(
        body,
        grid=(N_PAD // _RB,),
        in_specs=[
            pl.BlockSpec((_RB, H), lambda i: (i, 0)),
            pl.BlockSpec((NC, _RB, H), lambda i: (0, i, 0)),
            pl.BlockSpec((_RB, 8), lambda i: (i, 0)),
            pl.BlockSpec((1, 2 * H), lambda i: (0, 0)),
            pl.BlockSpec((H, H), lambda i: (0, 0)),
        ],
        out_specs=out_specs,
        out_shape=out_shape,
    )(h, parts, dg, gb, wn)


_QB = 512


def _head_body(st_ref, bo_ref, ns_ref, qx_ref, wm_ref, bm_ref, wg1_ref,
               bg1_ref, wg2_ref, bg2_ref, wq_ref, bq_ref, wc_ref, bc_ref,
               wbw1_ref, bbw1_ref, wbw2_ref, bbw2_ref, out_ref):
    st = st_ref[...]
    bo = bo_ref[...]
    ns = ns_ref[...]
    wm = wm_ref[...]
    base = jax.nn.gelu(_ln(st @ wm[:H] + bo @ wm[H:] + bm_ref[...]))
    gate = jax.nn.sigmoid(
        jax.nn.gelu(ns @ wg1_ref[...] + bg1_ref[...]) @ wg2_ref[...]
        + bg2_ref[...])
    matchup = base * (1.0 + 0.1 * gate)
    query = jax.nn.gelu(_ln(qx_ref[...] @ wq_ref[...] + bq_ref[...]))
    wc = wc_ref[...]
    readout = jax.nn.gelu(_ln(matchup @ wc[:H] + query @ wc[H:] + bc_ref[...]))
    t = jax.nn.relu(readout @ wbw1_ref[...] + bbw1_ref[...])
    out_ref[...] = t @ wbw2_ref[...] + bbw2_ref[...]


def _head(st, bo, ns, qx, wm, bm, wg1, bg1, wg2, bg2, wq, bq, wc, bc,
          wbw1, bbw1, wbw2, bbw2):
    row = lambda i: (i, 0)
    full = lambda i: (0, 0)
    return pl.pallas_call(
        _head_body,
        grid=(B // _QB,),
        in_specs=[
            pl.BlockSpec((_QB, H), row),
            pl.BlockSpec((_QB, H), row),
            pl.BlockSpec((_QB, H), row),
            pl.BlockSpec((_QB, H), row),
            pl.BlockSpec((2 * H, H), full),
            pl.BlockSpec((1, H), full),
            pl.BlockSpec((H, H // 4), full),
            pl.BlockSpec((1, H // 4), full),
            pl.BlockSpec((H // 4, H), full),
            pl.BlockSpec((1, H), full),
            pl.BlockSpec((H, H), full),
            pl.BlockSpec((1, H), full),
            pl.BlockSpec((2 * H, H), full),
            pl.BlockSpec((1, H), full),
            pl.BlockSpec((H, H), full),
            pl.BlockSpec((1, H), full),
            pl.BlockSpec((H, 2), full),
            pl.BlockSpec((1, 2), full),
        ],
        out_specs=pl.BlockSpec((_QB, 2), row),
        out_shape=jax.ShapeDtypeStruct((B, 2), _f32),
    )(st, bo, ns, qx, wm, bm, wg1, bg1, wg2, bg2, wq, bq, wc, bc,
      wbw1, bbw1, wbw2, bbw2)


_mesh = plsc.VectorSubcoreMesh(core_axis_name="c", subcore_axis_name="s",
                               num_cores=NC, num_subcores=NS)


_NZB = ROWS_PER_S // CHUNK


def _edge_scratch(width):
    return [
        pltpu.VMEM_SHARED((N_PAD, width), _f32),
        pltpu.VMEM((PF, CHUNK), jnp.int32),
        pltpu.VMEM((PF, CHUNK), jnp.int32),
        pltpu.VMEM((CHUNK, width), _f32),
        pltpu.VMEM((CHUNK, width), _f32),
        pltpu.SemaphoreType.DMA,
        pltpu.SemaphoreType.DMA,
    ]


def _make_edge(width):
    def body(hw_hbm, src3_hbm, dst3_hbm, zrow_hbm, out_hbm,
             agg_sh, srcs_v, dsts_v, rows0, rows1, sem0, sem1):
        c = lax.axis_index("c")
        s = lax.axis_index("s")
        w = s * NC + c
        r0 = s * ROWS_PER_S
        pltpu.sync_copy(zrow_hbm, rows0)
        for k in range(_NZB):
            pltpu.sync_copy(rows0, agg_sh.at[pl.ds(r0 + k * CHUNK, CHUNK)])
        plsc.subcore_barrier()
        rows = (rows0, rows1)
        sems = (sem0, sem1)

        def block(p, carry):
            pltpu.sync_copy(src3_hbm.at[w, p], srcs_v)
            pltpu.sync_copy(dst3_hbm.at[w, p], dsts_v)
            pltpu.async_copy(hw_hbm.at[srcs_v.at[0]], rows0, sem0)

            @pl.loop(0, PF - 1, step=2)
            def _pair(j2):
                for b in range(2):
                    j = j2 + b
                    pltpu.async_copy(hw_hbm.at[srcs_v.at[j + 1]],
                                     rows[1 - b], sems[1 - b])
                    pltpu.make_async_copy(hw_hbm.at[srcs_v.at[j]],
                                          rows[b], sems[b]).wait()
                    pltpu.sync_copy(rows[b], agg_sh.at[dsts_v.at[j]],
                                    add=True)

            pltpu.make_async_copy(hw_hbm.at[srcs_v.at[PF - 1]],
                                  rows0, sem0).wait()
            pltpu.sync_copy(rows0, agg_sh.at[dsts_v.at[PF - 1]], add=True)
            return carry

        lax.fori_loop(0, NBLK, block, 0)
        plsc.subcore_barrier()
        for k in range(_NZB):
            pltpu.sync_copy(agg_sh.at[pl.ds(r0 + k * CHUNK, CHUNK)], rows0)
            pltpu.sync_copy(rows0, out_hbm.at[c, pl.ds(r0 + k * CHUNK, CHUNK)])

    return pl.kernel(
        body,
        out_type=jax.ShapeDtypeStruct((NC, N_PAD, width), _f32),
        mesh=_mesh,
        scratch_types=_edge_scratch(width),
    )


_edge = _make_edge(H)


DW = 128


def _deg_body(dst3_hbm, ones_hbm, zdeg_hbm, out_hbm, agg_sh, dsts_v, rows_v):
    c = lax.axis_index("c")
    s = lax.axis_index("s")
    w = s * NC + c
    r0 = s * ROWS_PER_S
    pltpu.sync_copy(zdeg_hbm, rows_v)
    for k in range(_NZB):
        pltpu.sync_copy(rows_v, agg_sh.at[pl.ds(r0 + k * CHUNK, CHUNK)])
    pltpu.sync_copy(ones_hbm, rows_v)
    plsc.subcore_barrier()

    def block(p, carry):
        pltpu.sync_copy(dst3_hbm.at[w, p], dsts_v)

        @pl.loop(0, PF)
        def _step(i):
            pltpu.sync_copy(rows_v, agg_sh.at[dsts_v.at[i]], add=True)

        return carry

    lax.fori_loop(0, NBLK, block, 0)
    plsc.subcore_barrier()
    for k in range(_NZB):
        pltpu.sync_copy(agg_sh.at[pl.ds(r0 + k * CHUNK, CHUNK)], rows_v)
        pltpu.sync_copy(rows_v, out_hbm.at[c, pl.ds(r0 + k * CHUNK, CHUNK)])


_deg = pl.kernel(
    _deg_body,
    out_type=jax.ShapeDtypeStruct((NC, N_PAD, DW), _f32),
    mesh=_mesh,
    scratch_types=[
        pltpu.VMEM_SHARED((N_PAD, DW), _f32),
        pltpu.VMEM((PF, CHUNK), jnp.int32),
        pltpu.VMEM((CHUNK, DW), _f32),
    ],
)


_GPW = (3 * B) // NW
_GCH = 128


def _gather_body(h_hbm, ids_hbm, out_hbm, idx_v, rows_v, sem):
    c = lax.axis_index("c")
    s = lax.axis_index("s")
    w = s * NC + c
    for j in range(_GPW // _GCH):
        base = pl.multiple_of(w * _GPW + j * _GCH, 8)
        pltpu.sync_copy(ids_hbm.at[pl.ds(base, _GCH)], idx_v)
        pltpu.async_copy(h_hbm.at[idx_v], rows_v, sem).wait()
        pltpu.sync_copy(rows_v, out_hbm.at[pl.ds(base, _GCH)])


_gather = pl.kernel(
    _gather_body,
    out_type=jax.ShapeDtypeStruct((3 * B, H), _f32),
    mesh=_mesh,
    scratch_types=[
        pltpu.VMEM((_GCH,), jnp.int32),
        pltpu.VMEM((_GCH, H), _f32),
        pltpu.SemaphoreType.DMA,
    ],
)


def kernel(query_x, phase_state, chase_state, wicket_buffer, player_table,
           W_enc, b_enc, W_msg, W_film, b_film, W_m, b_m, W_g1, b_g1, W_g2,
           b_g2, W_q, b_q, W_c, b_c, W_b1, b_b1, W_b2, b_b2, W_w1, b_w1,
           W_w2, b_w2, striker_ids, bowler_ids, nonstriker_ids, edge_index):
    src = edge_index[0]
    dst = edge_index[1]

    pt_pad = jnp.pad(player_table, ((0, N_PAD - N_NODES), (0, 0)))
    h, hw, gb = _encode(pt_pad, W_enc, b_enc.reshape(1, -1), W_msg[0],
                        phase_state, chase_state, wicket_buffer, W_film,
                        b_film.reshape(1, -1))

    zrow = jnp.zeros((CHUNK, H), _f32)
    zdeg = jnp.zeros((CHUNK, DW), _f32)
    ones = jnp.ones((CHUNK, DW), _f32)

    src3 = src.reshape(NW, NBLK, PF, CHUNK)
    dst3 = dst.reshape(NW, NBLK, PF, CHUNK)
    degparts = _deg(dst3, ones, zdeg)
    parts = _edge(hw, src3, dst3, zrow)
    h, hw, dg = _update1(h, parts, degparts, gb, W_msg[1])
    parts = _edge(hw, src3, dst3, zrow)
    h, hw = _update(h, parts, dg, gb, W_msg[2])
    parts = _edge(hw, src3, dst3, zrow)
    h = _update(h, parts, dg, gb, None)[0]

    ids = jnp.concatenate([striker_ids, bowler_ids, nonstriker_ids])
    gath = _gather(h, ids.astype(jnp.int32))
    st, bo, ns = gath[:B], gath[B:2 * B], gath[2 * B:]

    wbw1 = jnp.concatenate([W_b1, W_w1], axis=1)
    bbw1 = jnp.concatenate([b_b1, b_w1]).reshape(1, -1)
    wbw2 = jnp.zeros((H, 2), _f32)
    wbw2 = wbw2.at[:H // 2, 0:1].set(W_b2)
    wbw2 = wbw2.at[H // 2:, 1:2].set(W_w2)
    bbw2 = jnp.concatenate([b_b2, b_w2]).reshape(1, -1)

    return _head(st, bo, ns, query_x, W_m, b_m.reshape(1, -1),
                 W_g1, b_g1.reshape(1, -1), W_g2, b_g2.reshape(1, -1),
                 W_q, b_q.reshape(1, -1), W_c, b_c.reshape(1, -1),
                 wbw1, bbw1, wbw2, bbw2)


# speedup vs baseline: 8.0573x; 1.0245x over previous
import functools

import jax
import jax.numpy as jnp
from jax import lax
from jax.experimental import pallas as pl
from jax.experimental.pallas import tpu as pltpu
from jax.experimental.pallas import tpu_sc as plsc

N_NODES = 10000
N_PAD = 10240
N_EDGES = 320000
B = 4096
H = 128
PED = 64

NC = 2
NS = 16
NW = NC * NS
EDGES_PER_W = N_EDGES // NW
CHUNK = 80
NCHUNK = EDGES_PER_W // CHUNK
PF = 25
NBLK = NCHUNK // PF
ROWS_PER_S = N_PAD // NS

_f32 = jnp.float32


def _ln(x, eps=1e-5):
    mu = jnp.mean(x, axis=-1, keepdims=True)
    var = jnp.var(x, axis=-1, keepdims=True)
    return (x - mu) / jnp.sqrt(var + eps)


_RB = 2048


def _encode_body(pt_ref, we_ref, be_ref, w0_ref, ph_ref, ch_ref, wk_ref,
                 wf_ref, bf_ref, h_ref, hw0_ref, gb_ref):
    x = pt_ref[...] @ we_ref[...] + be_ref[...]
    h = jax.nn.gelu(_ln(x))
    h_ref[...] = h
    hw0_ref[...] = h @ w0_ref[0]
    cond = jnp.concatenate([ph_ref[...], ch_ref[...], wk_ref[...]], axis=1)
    f = jnp.sum(cond, axis=0, keepdims=True) * (1.0 / B)
    gb_ref[...] = f @ wf_ref[...] + bf_ref[...]


def _encode(pt, we, be, w0, ph, ch, wk, wf, bf):
    return pl.pallas_call(
        _encode_body,
        grid=(N_PAD // _RB,),
        in_specs=[
            pl.BlockSpec((_RB, PED), lambda i: (i, 0)),
            pl.BlockSpec((PED, H), lambda i: (0, 0)),
            pl.BlockSpec((1, H), lambda i: (0, 0)),
            pl.BlockSpec((1, H, H), lambda i: (0, 0, 0)),
            pl.BlockSpec((B, 6), lambda i: (0, 0)),
            pl.BlockSpec((B, 7), lambda i: (0, 0)),
            pl.BlockSpec((B, 2), lambda i: (0, 0)),
            pl.BlockSpec((15, 2 * H), lambda i: (0, 0)),
            pl.BlockSpec((1, 2 * H), lambda i: (0, 0)),
        ],
        out_specs=[
            pl.BlockSpec((_RB, H), lambda i: (i, 0)),
            pl.BlockSpec((_RB, H), lambda i: (i, 0)),
            pl.BlockSpec((1, 2 * H), lambda i: (0, 0)),
        ],
        out_shape=[
            jax.ShapeDtypeStruct((N_PAD, H), _f32),
            jax.ShapeDtypeStruct((N_PAD, H), _f32),
            jax.ShapeDtypeStruct((1, 2 * H), _f32),
        ],
    )(pt, we, be, w0, ph, ch, wk, wf, bf)


def _update1_body(h_ref, p_ref, dp_ref, gb_ref, wn_ref,
                  hn_ref, hwn_ref, dg_ref):
    deg = jnp.maximum(dp_ref[0, :, :1] + dp_ref[1, :, :1], 1.0)
    agg = (p_ref[0] + p_ref[1]) / deg
    gb = gb_ref[...]
    u = jax.nn.gelu(agg * (1.0 + gb[:, :H]) + gb[:, H:])
    hn = _ln(h_ref[...] + u)
    hn_ref[...] = hn
    hwn_ref[...] = hn @ wn_ref[0]
    dg_ref[...] = jnp.broadcast_to(deg, (_RB, 8))


def _update1(h, parts, degparts, gb, wn):
    return pl.pallas_call(
        _update1_body,
        grid=(N_PAD // _RB,),
        in_specs=[
            pl.BlockSpec((_RB, H), lambda i: (i, 0)),
            pl.BlockSpec((NC, _RB, H), lambda i: (0, i, 0)),
            pl.BlockSpec((NC, _RB, DW), lambda i: (0, i, 0)),
            pl.BlockSpec((1, 2 * H), lambda i: (0, 0)),
            pl.BlockSpec((1, H, H), lambda i: (1, 0, 0)),
        ],
        out_specs=[
            pl.BlockSpec((_RB, H), lambda i: (i, 0)),
            pl.BlockSpec((_RB, H), lambda i: (i, 0)),
            pl.BlockSpec((_RB, 8), lambda i: (i, 0)),
        ],
        out_shape=[
            jax.ShapeDtypeStruct((N_PAD, H), _f32),
            jax.ShapeDtypeStruct((N_PAD, H), _f32),
            jax.ShapeDtypeStruct((N_PAD, 8), _f32),
        ],
    )(h, parts, degparts, gb, wn)


def _update_body(with_next, h_ref, p_ref, dg_ref, gb_ref,
                 wn_ref, hn_ref, hwn_ref=None):
    agg = (p_ref[0] + p_ref[1]) / dg_ref[...][:, :1]
    gb = gb_ref[...]
    u = jax.nn.gelu(agg * (1.0 + gb[:, :H]) + gb[:, H:])
    hn = _ln(h_ref[...] + u)
    hn_ref[...] = hn
    if with_next:
        hwn_ref[...] = hn @ wn_ref[0]


def _update(h, parts, dg, gb, wn, wl):
    with_next = wn is not None
    if not with_next:
        wn = jnp.zeros((1, H, H), _f32)
    body = functools.partial(_update_body, with_next)
    out_specs = [pl.BlockSpec((_RB, H), lambda i: (i, 0))]
    out_shape = [jax.ShapeDtypeStruct((N_PAD, H), _f32)]
    if with_next:
        out_specs.append(pl.BlockSpec((_RB, H), lambda i: (i, 0)))
        out_shape.append(jax.ShapeDtypeStruct((N_PAD, H), _f32))
    return pl.pallas_call(
        body,
        grid=(N_PAD // _RB,),
        in_specs=[
            pl.BlockSpec((_RB, H), lambda i: (i, 0)),
            pl.BlockSpec((NC, _RB, H), lambda i: (0, i, 0)),
            pl.BlockSpec((_RB, 8), lambda i: (i, 0)),
            pl.BlockSpec((1, 2 * H), lambda i: (0, 0)),
            pl.BlockSpec((1, H, H), lambda i: (wl, 0, 0)),
        ],
        out_specs=out_specs,
        out_shape=out_shape,
    )(h, parts, dg, gb, wn)


_QB = 512


def _head_body(st_ref, bo_ref, ns_ref, qx_ref, wm_ref, bm_ref, wg1_ref,
               bg1_ref, wg2_ref, bg2_ref, wq_ref, bq_ref, wc_ref, bc_ref,
               wbw1_ref, bbw1_ref, wbw2_ref, bbw2_ref, out_ref):
    st = st_ref[...]
    bo = bo_ref[...]
    ns = ns_ref[...]
    wm = wm_ref[...]
    base = jax.nn.gelu(_ln(st @ wm[:H] + bo @ wm[H:] + bm_ref[...]))
    gate = jax.nn.sigmoid(
        jax.nn.gelu(ns @ wg1_ref[...] + bg1_ref[...]) @ wg2_ref[...]
        + bg2_ref[...])
    matchup = base * (1.0 + 0.1 * gate)
    query = jax.nn.gelu(_ln(qx_ref[...] @ wq_ref[...] + bq_ref[...]))
    wc = wc_ref[...]
    readout = jax.nn.gelu(_ln(matchup @ wc[:H] + query @ wc[H:] + bc_ref[...]))
    t = jax.nn.relu(readout @ wbw1_ref[...] + bbw1_ref[...])
    out_ref[...] = t @ wbw2_ref[...] + bbw2_ref[...]


def _head(st, bo, ns, qx, wm, bm, wg1, bg1, wg2, bg2, wq, bq, wc, bc,
          wbw1, bbw1, wbw2, bbw2):
    row = lambda i: (i, 0)
    full = lambda i: (0, 0)
    return pl.pallas_call(
        _head_body,
        grid=(B // _QB,),
        in_specs=[
            pl.BlockSpec((_QB, H), row),
            pl.BlockSpec((_QB, H), row),
            pl.BlockSpec((_QB, H), row),
            pl.BlockSpec((_QB, H), row),
            pl.BlockSpec((2 * H, H), full),
            pl.BlockSpec((1, H), full),
            pl.BlockSpec((H, H // 4), full),
            pl.BlockSpec((1, H // 4), full),
            pl.BlockSpec((H // 4, H), full),
            pl.BlockSpec((1, H), full),
            pl.BlockSpec((H, H), full),
            pl.BlockSpec((1, H), full),
            pl.BlockSpec((2 * H, H), full),
            pl.BlockSpec((1, H), full),
            pl.BlockSpec((H, H), full),
            pl.BlockSpec((1, H), full),
            pl.BlockSpec((H, 2), full),
            pl.BlockSpec((1, 2), full),
        ],
        out_specs=pl.BlockSpec((_QB, 2), row),
        out_shape=jax.ShapeDtypeStruct((B, 2), _f32),
    )(st, bo, ns, qx, wm, bm, wg1, bg1, wg2, bg2, wq, bq, wc, bc,
      wbw1, bbw1, wbw2, bbw2)


_mesh = plsc.VectorSubcoreMesh(core_axis_name="c", subcore_axis_name="s",
                               num_cores=NC, num_subcores=NS)


def _edge_scratch(width):
    return [
        pltpu.VMEM_SHARED((N_PAD, width), _f32),
        pltpu.VMEM((PF, CHUNK), jnp.int32),
        pltpu.VMEM((PF, CHUNK), jnp.int32),
        pltpu.VMEM((CHUNK, width), _f32),
        pltpu.VMEM((CHUNK, width), _f32),
        pltpu.SemaphoreType.DMA,
        pltpu.SemaphoreType.DMA,
    ]


def _make_edge(width):
    def body(hw_hbm, ei5_hbm, zrow_hbm, out_hbm,
             agg_sh, srcs_v, dsts_v, rows0, rows1, sem0, sem1):
        c = lax.axis_index("c")
        s = lax.axis_index("s")
        w = s * NC + c
        r0 = s * ROWS_PER_S
        pltpu.sync_copy(zrow_hbm.at[pl.ds(r0, ROWS_PER_S)],
                        agg_sh.at[pl.ds(r0, ROWS_PER_S)])
        plsc.subcore_barrier()
        rows = (rows0, rows1)
        sems = (sem0, sem1)

        def block(p, carry):
            pltpu.sync_copy(ei5_hbm.at[0, w, p], srcs_v)
            pltpu.sync_copy(ei5_hbm.at[1, w, p], dsts_v)
            pltpu.async_copy(hw_hbm.at[srcs_v.at[0]], rows0, sem0)

            @pl.loop(0, PF - 1, step=2)
            def _pair(j2):
                for b in range(2):
                    j = j2 + b
                    pltpu.async_copy(hw_hbm.at[srcs_v.at[j + 1]],
                                     rows[1 - b], sems[1 - b])
                    pltpu.make_async_copy(hw_hbm.at[srcs_v.at[j]],
                                          rows[b], sems[b]).wait()
                    pltpu.sync_copy(rows[b], agg_sh.at[dsts_v.at[j]],
                                    add=True)

            pltpu.make_async_copy(hw_hbm.at[srcs_v.at[PF - 1]],
                                  rows0, sem0).wait()
            pltpu.sync_copy(rows0, agg_sh.at[dsts_v.at[PF - 1]], add=True)
            return carry

        lax.fori_loop(0, NBLK, block, 0)
        plsc.subcore_barrier()
        pltpu.sync_copy(agg_sh.at[pl.ds(r0, ROWS_PER_S)],
                        out_hbm.at[c, pl.ds(r0, ROWS_PER_S)])

    return pl.kernel(
        body,
        out_type=jax.ShapeDtypeStruct((NC, N_PAD, width), _f32),
        mesh=_mesh,
        scratch_types=_edge_scratch(width),
    )


_edge = _make_edge(H)


DW = 128


def _deg_body(ei5_hbm, ones_hbm, zdeg_hbm, out_hbm, agg_sh, dsts_v, rows_v):
    c = lax.axis_index("c")
    s = lax.axis_index("s")
    w = s * NC + c
    r0 = s * ROWS_PER_S
    pltpu.sync_copy(zdeg_hbm.at[pl.ds(r0, ROWS_PER_S)],
                    agg_sh.at[pl.ds(r0, ROWS_PER_S)])
    pltpu.sync_copy(ones_hbm, rows_v)
    plsc.subcore_barrier()

    def block(p, carry):
        pltpu.sync_copy(ei5_hbm.at[1, w, p], dsts_v)

        @pl.loop(0, PF)
        def _step(i):
            pltpu.sync_copy(rows_v, agg_sh.at[dsts_v.at[i]], add=True)

        return carry

    lax.fori_loop(0, NBLK, block, 0)
    plsc.subcore_barrier()
    pltpu.sync_copy(agg_sh.at[pl.ds(r0, ROWS_PER_S)],
                    out_hbm.at[c, pl.ds(r0, ROWS_PER_S)])


_deg = pl.kernel(
    _deg_body,
    out_type=jax.ShapeDtypeStruct((NC, N_PAD, DW), _f32),
    mesh=_mesh,
    scratch_types=[
        pltpu.VMEM_SHARED((N_PAD, DW), _f32),
        pltpu.VMEM((PF, CHUNK), jnp.int32),
        pltpu.VMEM((CHUNK, DW), _f32),
    ],
)


_GPW = (3 * B) // NW
_GCH = 128


def _gather_body(h_hbm, ids_hbm, out_hbm, idx_v, rows_v, sem):
    c = lax.axis_index("c")
    s = lax.axis_index("s")
    w = s * NC + c
    for j in range(_GPW // _GCH):
        base = pl.multiple_of(w * _GPW + j * _GCH, 8)
        pltpu.sync_copy(ids_hbm.at[pl.ds(base, _GCH)], idx_v)
        pltpu.async_copy(h_hbm.at[idx_v], rows_v, sem).wait()
        pltpu.sync_copy(rows_v, out_hbm.at[pl.ds(base, _GCH)])


_gather = pl.kernel(
    _gather_body,
    out_type=jax.ShapeDtypeStruct((3 * B, H), _f32),
    mesh=_mesh,
    scratch_types=[
        pltpu.VMEM((_GCH,), jnp.int32),
        pltpu.VMEM((_GCH, H), _f32),
        pltpu.SemaphoreType.DMA,
    ],
)


def kernel(query_x, phase_state, chase_state, wicket_buffer, player_table,
           W_enc, b_enc, W_msg, W_film, b_film, W_m, b_m, W_g1, b_g1, W_g2,
           b_g2, W_q, b_q, W_c, b_c, W_b1, b_b1, W_b2, b_b2, W_w1, b_w1,
           W_w2, b_w2, striker_ids, bowler_ids, nonstriker_ids, edge_index):

    pt_pad = jnp.pad(player_table, ((0, N_PAD - N_NODES), (0, 0)))
    h, hw, gb = _encode(pt_pad, W_enc, b_enc.reshape(1, -1), W_msg,
                        phase_state, chase_state, wicket_buffer, W_film,
                        b_film.reshape(1, -1))

    zrow = jnp.zeros((N_PAD, H), _f32)
    zdeg = jnp.zeros((N_PAD, DW), _f32)
    ones = jnp.ones((CHUNK, DW), _f32)

    ei5 = edge_index.reshape(2, NW, NBLK, PF, CHUNK)
    degparts = _deg(ei5, ones, zdeg)
    parts = _edge(hw, ei5, zrow)
    h, hw, dg = _update1(h, parts, degparts, gb, W_msg)
    parts = _edge(hw, ei5, zrow)
    h, hw = _update(h, parts, dg, gb, W_msg, 2)
    parts = _edge(hw, ei5, zrow)
    h = _update(h, parts, dg, gb, None, 0)[0]

    ids = jnp.concatenate([striker_ids, bowler_ids, nonstriker_ids])
    gath = _gather(h, ids.astype(jnp.int32))
    st, bo, ns = gath[:B], gath[B:2 * B], gath[2 * B:]

    wbw1 = jnp.concatenate([W_b1, W_w1], axis=1)
    bbw1 = jnp.concatenate([b_b1, b_w1]).reshape(1, -1)
    wbw2 = jnp.zeros((H, 2), _f32)
    wbw2 = wbw2.at[:H // 2, 0:1].set(W_b2)
    wbw2 = wbw2.at[H // 2:, 1:2].set(W_w2)
    bbw2 = jnp.concatenate([b_b2, b_w2]).reshape(1, -1)

    return _head(st, bo, ns, query_x, W_m, b_m.reshape(1, -1),
                 W_g1, b_g1.reshape(1, -1), W_g2, b_g2.reshape(1, -1),
                 W_q, b_q.reshape(1, -1), W_c, b_c.reshape(1, -1),
                 wbw1, bbw1, wbw2, bbw2)
